# baseline (speedup 1.0000x reference)
import jax
import jax.numpy as jnp
from jax.experimental import pallas as pl
from jax.experimental.pallas import tpu as pltpu


def _concat_body(pp, cp, sp, pn, cn, sn, tp, tn, out3, out_t, buf, sem):
    dim = pp.shape[1]
    n_ctx = cp.shape[1] // dim
    suf = sp.shape[0]
    head = 1 + n_ctx
    buf[0, 0:1, :] = pp[...]
    buf[0, 1:2, :] = pn[...]
    for r in range(n_ctx):
        buf[1 + r, 0:1, :] = cp[0:1, r * dim:(r + 1) * dim]
        buf[1 + r, 1:2, :] = cn[0:1, r * dim:(r + 1) * dim]
    head_cp = pltpu.make_async_copy(
        buf.at[pl.ds(0, head)], out3.at[pl.ds(0, head)], sem)
    head_cp.start()
    buf[head:head + suf, 0:1, :] = sp[...].reshape(suf, 1, dim)
    buf[head:head + suf, 1:2, :] = sn[...].reshape(suf, 1, dim)
    tail_cp = pltpu.make_async_copy(
        buf.at[pl.ds(head, suf)], out3.at[pl.ds(head, suf)], sem)
    tail_cp.start()
    out_t[0:1, :] = tp[...]
    out_t[1:2, :] = tn[...]
    head_cp.wait()
    tail_cp.wait()


def kernel(ctx_pos, ctx_neg, token_prefix_pos, token_suffix_pos,
           token_prefix_neg, token_suffix_neg,
           tokenized_prompts_pos, tokenized_prompts_neg, cls_id):
    n_ctx = ctx_pos.shape[2]
    dim = ctx_pos.shape[3]
    suf = token_suffix_pos.shape[2]
    ctx_len = 1 + n_ctx + suf
    pp = token_prefix_pos.reshape(1, dim)
    cp = ctx_pos.reshape(1, n_ctx * dim)
    sp = token_suffix_pos.reshape(suf, dim)
    pn = token_prefix_neg.reshape(1, dim)
    cn = ctx_neg.reshape(1, n_ctx * dim)
    sn = token_suffix_neg.reshape(suf, dim)
    tp = tokenized_prompts_pos.reshape(1, ctx_len)
    tn = tokenized_prompts_neg.reshape(1, ctx_len)

    vmem = pl.BlockSpec(memory_space=pltpu.VMEM)
    out3, out_t = pl.pallas_call(
        _concat_body,
        in_specs=[vmem] * 8,
        out_specs=(pl.BlockSpec(memory_space=pl.ANY), vmem),
        out_shape=(
            jax.ShapeDtypeStruct((ctx_len, 2, dim), jnp.float32),
            jax.ShapeDtypeStruct((2, ctx_len), jnp.int32),
        ),
        scratch_shapes=[
            pltpu.VMEM((ctx_len, 2, dim), jnp.float32),
            pltpu.SemaphoreType.DMA,
        ],
    )(pp, cp, sp, pn, cn, sn, tp, tn)
    return out3.transpose(1, 0, 2), out_t


# speedup vs baseline: 1.1916x; 1.1916x over previous
import jax
import jax.numpy as jnp
from jax.experimental import pallas as pl


def _concat_body(pp, cp, sp, pn, cn, sn, tp, tn, out3, out_t):
    dim = pp.shape[1]
    n_ctx = cp.shape[1] // dim
    suf = sp.shape[0]
    out3[0, 0:1, :] = pp[...]
    out3[0, 1:2, :] = pn[...]
    for r in range(n_ctx):
        out3[1 + r, 0:1, :] = cp[0:1, r * dim:(r + 1) * dim]
        out3[1 + r, 1:2, :] = cn[0:1, r * dim:(r + 1) * dim]
    out3[1 + n_ctx:1 + n_ctx + suf, 0:1, :] = sp[...].reshape(suf, 1, dim)
    out3[1 + n_ctx:1 + n_ctx + suf, 1:2, :] = sn[...].reshape(suf, 1, dim)
    out_t[0:1, :] = tp[...]
    out_t[1:2, :] = tn[...]


def kernel(ctx_pos, ctx_neg, token_prefix_pos, token_suffix_pos,
           token_prefix_neg, token_suffix_neg,
           tokenized_prompts_pos, tokenized_prompts_neg, cls_id):
    n_ctx = ctx_pos.shape[2]
    dim = ctx_pos.shape[3]
    suf = token_suffix_pos.shape[2]
    ctx_len = 1 + n_ctx + suf
    pp = token_prefix_pos.reshape(1, dim)
    cp = ctx_pos.reshape(1, n_ctx * dim)
    sp = token_suffix_pos.reshape(suf, dim)
    pn = token_prefix_neg.reshape(1, dim)
    cn = ctx_neg.reshape(1, n_ctx * dim)
    sn = token_suffix_neg.reshape(suf, dim)
    tp = tokenized_prompts_pos.reshape(1, ctx_len)
    tn = tokenized_prompts_neg.reshape(1, ctx_len)

    out3, out_t = pl.pallas_call(
        _concat_body,
        out_shape=(
            jax.ShapeDtypeStruct((ctx_len, 2, dim), jnp.float32),
            jax.ShapeDtypeStruct((2, ctx_len), jnp.int32),
        ),
    )(pp, cp, sp, pn, cn, sn, tp, tn)
    return out3.transpose(1, 0, 2), out_t
